# initial kernel scaffold (unmeasured)
import jax
import jax.numpy as jnp
from jax import lax
from jax.experimental import pallas as pl
from jax.experimental.pallas import tpu as pltpu

N_DEV = 4
FP8 = jnp.float8_e4m3fn


def _ag_body(x_ref, w_ref, xg_ref, wg_ref, x_send, x_recv, w_send, w_recv):
    me = lax.axis_index("i")
    left = (me - 1) % N_DEV
    right = (me + 1) % N_DEV

    barrier = pltpu.get_barrier_semaphore()
    for nbr in (left, right):
        pl.semaphore_signal(
            barrier, inc=1, device_id=(nbr,),
            device_id_type=pl.DeviceIdType.MESH,
        )
    pl.semaphore_wait(barrier, 2)

    kx = x_ref.shape[1]
    kw = w_ref.shape[0]

    xg_ref[:, pl.ds(me * kx, kx)] = x_ref[...]
    wg_ref[pl.ds(me * kw, kw), :] = w_ref[...]

    for h in range(N_DEV - 1):
        origin = (me - h) % N_DEV
        rx = pltpu.make_async_remote_copy(
            src_ref=xg_ref.at[:, pl.ds(origin * kx, kx)],
            dst_ref=xg_ref.at[:, pl.ds(origin * kx, kx)],
            send_sem=x_send.at[h],
            recv_sem=x_recv.at[h],
            device_id=(right,),
            device_id_type=pl.DeviceIdType.MESH,
        )
        rw = pltpu.make_async_remote_copy(
            src_ref=wg_ref.at[pl.ds(origin * kw, kw), :],
            dst_ref=wg_ref.at[pl.ds(origin * kw, kw), :],
            send_sem=w_send.at[h],
            recv_sem=w_recv.at[h],
            device_id=(right,),
            device_id_type=pl.DeviceIdType.MESH,
        )
        rx.start()
        rw.start()
        rx.wait()
        rw.wait()


def _gemm_body(xg_ref, wg_ref, s_ref, o_ref):
    acc = jnp.dot(xg_ref[...], wg_ref[...], preferred_element_type=jnp.float32)
    o_ref[...] = jnp.maximum(acc * s_ref[0, 0], 0.0)


def kernel(x, w_mat, scale_x, scale_w):
    if x.dtype != FP8:
        x = x.astype(FP8)
    if w_mat.dtype != FP8:
        w_mat = w_mat.astype(FP8)
    s = (scale_x.astype(jnp.float32) * scale_w.astype(jnp.float32)).reshape(1, 1)

    m, kx = x.shape
    kw, n = w_mat.shape
    k = kx * N_DEV

    xg, wg = pl.pallas_call(
        _ag_body,
        out_shape=[
            jax.ShapeDtypeStruct((m, k), FP8),
            jax.ShapeDtypeStruct((k, n), FP8),
        ],
        in_specs=[
            pl.BlockSpec(memory_space=pltpu.VMEM),
            pl.BlockSpec(memory_space=pltpu.VMEM),
        ],
        out_specs=[
            pl.BlockSpec(memory_space=pltpu.VMEM),
            pl.BlockSpec(memory_space=pltpu.VMEM),
        ],
        scratch_shapes=[
            pltpu.SemaphoreType.DMA((N_DEV - 1,)),
            pltpu.SemaphoreType.DMA((N_DEV - 1,)),
            pltpu.SemaphoreType.DMA((N_DEV - 1,)),
            pltpu.SemaphoreType.DMA((N_DEV - 1,)),
        ],
        compiler_params=pltpu.CompilerParams(collective_id=0),
    )(x, w_mat)

    bm = 512
    y = pl.pallas_call(
        _gemm_body,
        grid=(m // bm,),
        in_specs=[
            pl.BlockSpec((bm, k), lambda i: (i, 0)),
            pl.BlockSpec((k, n), lambda i: (0, 0)),
            pl.BlockSpec(memory_space=pltpu.SMEM),
        ],
        out_specs=pl.BlockSpec((bm, n), lambda i: (i, 0)),
        out_shape=jax.ShapeDtypeStruct((m, n), jnp.float32),
    )(xg, wg, s)
    return y


# baseline (device time: 617006 ns/iter reference)
import jax
import jax.numpy as jnp
from jax import lax
from jax.experimental import pallas as pl
from jax.experimental.pallas import tpu as pltpu

N_DEV = 4
FP8 = jnp.float8_e4m3fn


def _ag_body(x_ref, w_ref, xg_ref, wg_ref, x_send, x_recv, w_send, w_recv):
    me = lax.axis_index("i")
    left = (me - 1) % N_DEV
    right = (me + 1) % N_DEV

    barrier = pltpu.get_barrier_semaphore()
    for nbr in (left, right):
        pl.semaphore_signal(
            barrier, inc=1, device_id=(nbr,),
            device_id_type=pl.DeviceIdType.MESH,
        )
    pl.semaphore_wait(barrier, 2)

    kx = x_ref.shape[1]
    kw = w_ref.shape[0]

    xg_ref[:, pl.ds(me * kx, kx)] = x_ref[...]
    wg_ref[pl.ds(me * kw, kw), :] = w_ref[...]

    for h in range(N_DEV - 1):
        origin = (me - h) % N_DEV
        rx = pltpu.make_async_remote_copy(
            src_ref=xg_ref.at[:, pl.ds(origin * kx, kx)],
            dst_ref=xg_ref.at[:, pl.ds(origin * kx, kx)],
            send_sem=x_send.at[h],
            recv_sem=x_recv.at[h],
            device_id=(right,),
            device_id_type=pl.DeviceIdType.MESH,
        )
        rw = pltpu.make_async_remote_copy(
            src_ref=wg_ref.at[pl.ds(origin * kw, kw), :],
            dst_ref=wg_ref.at[pl.ds(origin * kw, kw), :],
            send_sem=w_send.at[h],
            recv_sem=w_recv.at[h],
            device_id=(right,),
            device_id_type=pl.DeviceIdType.MESH,
        )
        rx.start()
        rw.start()
        rx.wait()
        rw.wait()


def _gemm_body(xg_ref, wg_ref, s_ref, o_ref):
    acc = jnp.dot(xg_ref[...], wg_ref[...], preferred_element_type=jnp.float32)
    o_ref[...] = jnp.maximum(acc * s_ref[0, 0], 0.0)


def kernel(x, w_mat, scale_x, scale_w):
    if x.dtype != FP8:
        x = x.astype(FP8)
    if w_mat.dtype != FP8:
        w_mat = w_mat.astype(FP8)
    s = (scale_x.astype(jnp.float32) * scale_w.astype(jnp.float32)).reshape(1, 1)

    m, kx = x.shape
    kw, n = w_mat.shape
    k = kx * N_DEV

    xg, wg = pl.pallas_call(
        _ag_body,
        out_shape=[
            jax.ShapeDtypeStruct((m, k), FP8),
            jax.ShapeDtypeStruct((k, n), FP8),
        ],
        in_specs=[
            pl.BlockSpec(memory_space=pltpu.VMEM),
            pl.BlockSpec(memory_space=pltpu.VMEM),
        ],
        out_specs=[
            pl.BlockSpec(memory_space=pltpu.VMEM),
            pl.BlockSpec(memory_space=pltpu.VMEM),
        ],
        scratch_shapes=[
            pltpu.SemaphoreType.DMA((N_DEV - 1,)),
            pltpu.SemaphoreType.DMA((N_DEV - 1,)),
            pltpu.SemaphoreType.DMA((N_DEV - 1,)),
            pltpu.SemaphoreType.DMA((N_DEV - 1,)),
        ],
        compiler_params=pltpu.CompilerParams(
            collective_id=0, vmem_limit_bytes=100 * 1024 * 1024
        ),
    )(x, w_mat)

    bm, bn = 512, 2048
    y = pl.pallas_call(
        _gemm_body,
        grid=(n // bn, m // bm),
        in_specs=[
            pl.BlockSpec((bm, k), lambda i, j: (j, 0)),
            pl.BlockSpec((k, bn), lambda i, j: (0, i)),
            pl.BlockSpec(memory_space=pltpu.SMEM),
        ],
        out_specs=pl.BlockSpec((bm, bn), lambda i, j: (j, i)),
        out_shape=jax.ShapeDtypeStruct((m, n), jnp.float32),
        compiler_params=pltpu.CompilerParams(
            vmem_limit_bytes=100 * 1024 * 1024
        ),
    )(xg, wg, s)
    return y


# device time: 415189 ns/iter; 1.4861x vs baseline; 1.4861x over previous
import jax
import jax.numpy as jnp
from jax import lax
from jax.experimental import pallas as pl
from jax.experimental.pallas import tpu as pltpu

N_DEV = 4
FP8 = jnp.float8_e4m3fn


def _ag_body(x_ref, w_ref, xg_ref, wg_ref, sems):
    me = lax.axis_index("i")
    left = (me - 1) % N_DEV
    right = (me + 1) % N_DEV

    barrier = pltpu.get_barrier_semaphore()
    for nbr in (left, right):
        pl.semaphore_signal(
            barrier, inc=1, device_id=(nbr,),
            device_id_type=pl.DeviceIdType.MESH,
        )
    pl.semaphore_wait(barrier, 2)

    kx = x_ref.shape[1]
    kw = w_ref.shape[0]
    mh = x_ref.shape[0] // 2
    nh = w_ref.shape[1] // 2

    xg_ref[:, pl.ds(me * kx, kx)] = x_ref[...]
    wg_ref[pl.ds(me * kw, kw), :] = w_ref[...]

    for h in range(N_DEV - 1):
        o_cw = (me - h) % N_DEV
        o_ccw = (me + h) % N_DEV
        rdmas = []
        for s, (src, dst) in enumerate((
            (xg_ref.at[pl.ds(0, mh), pl.ds(o_cw * kx, kx)], right),
            (wg_ref.at[pl.ds(o_cw * kw, kw), pl.ds(0, nh)], right),
            (xg_ref.at[pl.ds(mh, mh), pl.ds(o_ccw * kx, kx)], left),
            (wg_ref.at[pl.ds(o_ccw * kw, kw), pl.ds(nh, nh)], left),
        )):
            r = pltpu.make_async_remote_copy(
                src_ref=src,
                dst_ref=src,
                send_sem=sems.at[2 * s, h],
                recv_sem=sems.at[2 * s + 1, h],
                device_id=(dst,),
                device_id_type=pl.DeviceIdType.MESH,
            )
            r.start()
            rdmas.append(r)
        for r in rdmas:
            r.wait()


def _gemm_body(xg_ref, wg_ref, s_ref, o_ref):
    acc = jnp.dot(xg_ref[...], wg_ref[...], preferred_element_type=jnp.float32)
    o_ref[...] = jnp.maximum(acc * s_ref[0, 0], 0.0)


def kernel(x, w_mat, scale_x, scale_w):
    if x.dtype != FP8:
        x = x.astype(FP8)
    if w_mat.dtype != FP8:
        w_mat = w_mat.astype(FP8)
    s = (scale_x.astype(jnp.float32) * scale_w.astype(jnp.float32)).reshape(1, 1)

    m, kx = x.shape
    kw, n = w_mat.shape
    k = kx * N_DEV

    xg, wg = pl.pallas_call(
        _ag_body,
        out_shape=[
            jax.ShapeDtypeStruct((m, k), FP8),
            jax.ShapeDtypeStruct((k, n), FP8),
        ],
        in_specs=[
            pl.BlockSpec(memory_space=pltpu.VMEM),
            pl.BlockSpec(memory_space=pltpu.VMEM),
        ],
        out_specs=[
            pl.BlockSpec(memory_space=pltpu.VMEM),
            pl.BlockSpec(memory_space=pltpu.VMEM),
        ],
        scratch_shapes=[
            pltpu.SemaphoreType.DMA((8, N_DEV - 1)),
        ],
        compiler_params=pltpu.CompilerParams(
            collective_id=0, vmem_limit_bytes=100 * 1024 * 1024
        ),
    )(x, w_mat)

    bm, bn = 512, 2048
    y = pl.pallas_call(
        _gemm_body,
        grid=(n // bn, m // bm),
        in_specs=[
            pl.BlockSpec((bm, k), lambda i, j: (j, 0)),
            pl.BlockSpec((k, bn), lambda i, j: (0, i)),
            pl.BlockSpec(memory_space=pltpu.SMEM),
        ],
        out_specs=pl.BlockSpec((bm, bn), lambda i, j: (j, i)),
        out_shape=jax.ShapeDtypeStruct((m, n), jnp.float32),
        compiler_params=pltpu.CompilerParams(
            vmem_limit_bytes=100 * 1024 * 1024
        ),
    )(xg, wg, s)
    return y
